# Initial kernel scaffold; baseline (speedup 1.0000x reference)
import jax
import jax.numpy as jnp
from jax.experimental import pallas as pl


def kernel(x, We):
    raise NotImplementedError("write your pallas kernel here")


# speedup vs baseline: 1.4771x; 1.4771x over previous
import functools
import math

import jax
import jax.numpy as jnp
import numpy as np
from jax import lax
from jax.experimental import pallas as pl
from jax.experimental.pallas import tpu as pltpu
from jax.experimental.pallas import tpu_sc as plsc

VOCAB = 100000
D = 768
B = 4
T = 8192
N_ROWS = B * T
SCALE = math.sqrt(D)
LANES = 16
CHUNK = 64


def _pe_table():
    positions = np.arange(T, dtype=np.float32)[:, None]
    i = np.arange(0, D, 2, dtype=np.float32)
    denominator = np.exp(i / D * math.log(10000.0))
    pe = np.zeros((T, D), dtype=np.float32)
    pe[:, 0::2] = np.sin(positions / denominator)
    pe[:, 1::2] = np.cos(positions / denominator)
    return pe


_PE = _pe_table()


def _make_sc_kernel():
    info = plsc.get_sparse_core_info()
    nc, ns = info.num_cores, info.num_subcores
    nw = nc * ns
    rows_per_w = N_ROWS // nw
    n_chunks = rows_per_w // CHUNK
    mesh = plsc.VectorSubcoreMesh(core_axis_name="c", subcore_axis_name="s")

    @functools.partial(
        pl.kernel,
        mesh=mesh,
        out_type=jax.ShapeDtypeStruct((N_ROWS, D), jnp.float32),
        scratch_types=[
            pltpu.VMEM((CHUNK,), jnp.int32),
            pltpu.VMEM((CHUNK, D), jnp.float32),
            pltpu.VMEM((CHUNK, D), jnp.float32),
            pltpu.SemaphoreType.DMA,
        ],
    )
    def k(x_hbm, we_hbm, pe_hbm, out_hbm, idx_v, rows_v, pe_v, sem):
        wid = lax.axis_index("s") * nc + lax.axis_index("c")
        base = wid * rows_per_w
        t0 = lax.rem(base, T)

        def chunk_body(ci, _):
            row0 = base + ci * CHUNK
            pltpu.sync_copy(x_hbm.at[pl.ds(row0, CHUNK)], idx_v)
            gather = pltpu.async_copy(we_hbm.at[idx_v], rows_v, sem)
            pltpu.sync_copy(pe_hbm.at[pl.ds(t0 + ci * CHUNK, CHUNK)], pe_v)
            gather.wait()

            def row_body(r, _):
                for j in range(D // LANES):
                    sl = pl.ds(j * LANES, LANES)
                    rows_v[r, sl] = rows_v[r, sl] * SCALE + pe_v[r, sl]
                return 0

            lax.fori_loop(0, CHUNK, row_body, 0)
            pltpu.sync_copy(rows_v, out_hbm.at[pl.ds(row0, CHUNK)])
            return 0

        lax.fori_loop(0, n_chunks, chunk_body, 0)

    return k


_sc_kernel = _make_sc_kernel()


@jax.jit
def kernel(x, We):
    pe = jnp.asarray(_PE)
    flat_idx = x.reshape(-1).astype(jnp.int32)
    out = _sc_kernel(flat_idx, We, pe)
    return out.reshape(B, T, D)


# speedup vs baseline: 1.9646x; 1.3301x over previous
import functools
import math

import jax
import jax.numpy as jnp
import numpy as np
from jax import lax
from jax.experimental import pallas as pl
from jax.experimental.pallas import tpu as pltpu
from jax.experimental.pallas import tpu_sc as plsc

VOCAB = 100000
D = 768
B = 4
T = 8192
N_ROWS = B * T
SCALE = math.sqrt(D)
LANES = 16
C = 16


def _pe_table():
    positions = np.arange(T, dtype=np.float32)[:, None]
    i = np.arange(0, D, 2, dtype=np.float32)
    denominator = np.exp(i / D * math.log(10000.0))
    pe = np.zeros((T, D), dtype=np.float32)
    pe[:, 0::2] = np.sin(positions / denominator)
    pe[:, 1::2] = np.cos(positions / denominator)
    return pe


_PE = _pe_table()


def _make_sc_kernel():
    info = plsc.get_sparse_core_info()
    nc, ns = info.num_cores, info.num_subcores
    nw = nc * ns
    t_per_w = T // nw
    n_tc = t_per_w // C
    mesh = plsc.VectorSubcoreMesh(core_axis_name="c", subcore_axis_name="s")

    @functools.partial(
        pl.kernel,
        mesh=mesh,
        out_type=jax.ShapeDtypeStruct((N_ROWS, D), jnp.float32),
        scratch_types=[
            pltpu.VMEM((B, t_per_w), jnp.int32),
            [pltpu.VMEM((C, D), jnp.float32)] * B,
            [pltpu.VMEM((C, D), jnp.float32)] * 2,
            [pltpu.SemaphoreType.DMA] * B,
            [pltpu.SemaphoreType.DMA] * B,
            [pltpu.SemaphoreType.DMA] * 2,
        ],
    )
    def k(x_hbm, we_hbm, pe_hbm, out_hbm, idx_all, rows, pes, sg, so, sp):
        wid = lax.axis_index("s") * nc + lax.axis_index("c")
        t0 = wid * t_per_w

        def idx_slice(b, tc):
            return idx_all.at[b, pl.ds(tc * C, C)]

        def gather(b, tc, q):
            return pltpu.make_async_copy(
                we_hbm.at[idx_slice(b, tc)], rows[q], sg[q])

        def out_copy(b, tc, q):
            return pltpu.make_async_copy(
                rows[q], out_hbm.at[pl.ds(b * T + t0 + tc * C, C)], so[q])

        def pe_copy(tc, ph):
            return pltpu.make_async_copy(
                pe_hbm.at[pl.ds(t0 + tc * C, C)], pes[ph], sp[ph])

        for b in range(B):
            pltpu.sync_copy(x_hbm.at[pl.ds(b * T + t0, t_per_w)],
                            idx_all.at[b])
        pe_copy(0, 0).start()
        gather(0, 0, 0).start()
        gather(1, 0, 1).start()

        def compute(q, ph):
            def row_body(r, _):
                for j in range(D // LANES):
                    sl = pl.ds(j * LANES, LANES)
                    rows[q][r, sl] = rows[q][r, sl] * SCALE + pes[ph][r, sl]
                return 0
            lax.fori_loop(0, C, row_body, 0)

        def tco_body(tco, _):
            for phase in range(2):
                tc = tco * 2 + phase
                for b in range(B):
                    gather(b, tc, b).wait()
                    if b == 0:
                        pe_copy(tc, phase).wait()

                        @pl.when(tc + 1 < n_tc)
                        def _():
                            pe_copy(tc + 1, 1 - phase).start()
                    compute(b, phase)
                    out_copy(b, tc, b).start()
                    if b < 2:
                        q = b + 2

                        @pl.when(tc > 0)
                        def _():
                            out_copy(q, tc - 1, q).wait()
                        gather(q, tc, q).start()
                    else:
                        q = b - 2

                        @pl.when(tc + 1 < n_tc)
                        def _():
                            out_copy(q, tc, q).wait()
                            gather(q, tc + 1, q).start()
            return 0

        lax.fori_loop(0, n_tc // 2, tco_body, 0)
        for b in range(B):
            out_copy(b, n_tc - 1, b).wait()

    return k


_sc_kernel = _make_sc_kernel()


@jax.jit
def kernel(x, We):
    pe = jnp.asarray(_PE)
    flat_idx = x.reshape(-1).astype(jnp.int32)
    out = _sc_kernel(flat_idx, We, pe)
    return out.reshape(B, T, D)


# speedup vs baseline: 2.2352x; 1.1377x over previous
import functools
import math

import jax
import jax.numpy as jnp
import numpy as np
from jax import lax
from jax.experimental import pallas as pl
from jax.experimental.pallas import tpu as pltpu
from jax.experimental.pallas import tpu_sc as plsc

VOCAB = 100000
D = 768
B = 4
T = 8192
N_ROWS = B * T
SCALE = math.sqrt(D)
LANES = 16
C = 16


def _pe_table():
    positions = np.arange(T, dtype=np.float32)[:, None]
    i = np.arange(0, D, 2, dtype=np.float32)
    denominator = np.exp(i / D * math.log(10000.0))
    pe = np.zeros((T, D), dtype=np.float32)
    pe[:, 0::2] = np.sin(positions / denominator)
    pe[:, 1::2] = np.cos(positions / denominator)
    return pe


_PE = _pe_table()


def _make_sc_kernel():
    info = plsc.get_sparse_core_info()
    nc, ns = info.num_cores, info.num_subcores
    nw = nc * ns
    t_per_w = T // nw
    n_tc = t_per_w // C
    mesh = plsc.VectorSubcoreMesh(core_axis_name="c", subcore_axis_name="s")

    @functools.partial(
        pl.kernel,
        mesh=mesh,
        out_type=jax.ShapeDtypeStruct((N_ROWS, D), jnp.float32),
        scratch_types=[
            pltpu.VMEM((B, t_per_w), jnp.int32),
            [[pltpu.VMEM((C, D), jnp.float32)] * B] * 2,
            [pltpu.VMEM((C, D), jnp.float32)] * 2,
            [[pltpu.SemaphoreType.DMA] * B] * 2,
            [[pltpu.SemaphoreType.DMA] * B] * 2,
            [pltpu.SemaphoreType.DMA] * 2,
        ],
    )
    def k(x_hbm, we_hbm, pe_hbm, out_hbm, idx_all, rows, pes, sg, so, sp):
        wid = lax.axis_index("s") * nc + lax.axis_index("c")
        t0 = wid * t_per_w

        def gather(b, tc, g):
            return pltpu.make_async_copy(
                we_hbm.at[idx_all.at[b, pl.ds(tc * C, C)]],
                rows[g][b], sg[g][b])

        def out_copy(b, tc, g):
            return pltpu.make_async_copy(
                rows[g][b], out_hbm.at[pl.ds(b * T + t0 + tc * C, C)],
                so[g][b])

        def pe_copy(tc, g):
            return pltpu.make_async_copy(
                pe_hbm.at[pl.ds(t0 + tc * C, C)], pes[g], sp[g])

        for b in range(B):
            pltpu.sync_copy(x_hbm.at[pl.ds(b * T + t0, t_per_w)],
                            idx_all.at[b])
        pe_copy(0, 0).start()
        for b in range(B):
            gather(b, 0, 0).start()

        def tco_body(tco, _):
            for g in range(2):
                tc = tco * 2 + g
                for b in range(B):
                    gather(b, tc, g).wait()
                pe_copy(tc, g).wait()

                @pl.when(tc + 1 < n_tc)
                def _():
                    pe_copy(tc + 1, 1 - g).start()

                @pl.when(tc > 0)
                def _():
                    for b in range(B):
                        out_copy(b, tc - 1, 1 - g).wait()

                @pl.when(tc + 1 < n_tc)
                def _():
                    for b in range(B):
                        gather(b, tc + 1, 1 - g).start()

                def row_body(r, _):
                    for j in range(D // LANES):
                        sl = pl.ds(j * LANES, LANES)
                        pe_vec = pes[g][r, sl]
                        for b in range(B):
                            rows[g][b][r, sl] = (
                                rows[g][b][r, sl] * SCALE + pe_vec)
                    return 0

                lax.fori_loop(0, C, row_body, 0)
                for b in range(B):
                    out_copy(b, tc, g).start()
            return 0

        lax.fori_loop(0, n_tc // 2, tco_body, 0)
        for b in range(B):
            out_copy(b, n_tc - 1, 1).wait()

    return k


_sc_kernel = _make_sc_kernel()


@jax.jit
def kernel(x, We):
    pe = jnp.asarray(_PE)
    flat_idx = x.reshape(-1).astype(jnp.int32)
    out = _sc_kernel(flat_idx, We, pe)
    return out.reshape(B, T, D)


# speedup vs baseline: 2.5100x; 1.1230x over previous
import functools
import math

import jax
import jax.numpy as jnp
import numpy as np
from jax import lax
from jax.experimental import pallas as pl
from jax.experimental.pallas import tpu as pltpu
from jax.experimental.pallas import tpu_sc as plsc

VOCAB = 100000
D = 768
B = 4
T = 8192
N_ROWS = B * T
SCALE = math.sqrt(D)
LANES = 16
C = 16


def _pe_table():
    positions = np.arange(T, dtype=np.float32)[:, None]
    i = np.arange(0, D, 2, dtype=np.float32)
    denominator = np.exp(i / D * math.log(10000.0))
    pe = np.zeros((T, D), dtype=np.float32)
    pe[:, 0::2] = np.sin(positions / denominator)
    pe[:, 1::2] = np.cos(positions / denominator)
    return pe


_PE = _pe_table()


def _make_sc_kernel():
    info = plsc.get_sparse_core_info()
    nc, ns = info.num_cores, info.num_subcores
    nw = nc * ns
    t_per_w = T // nw
    n_tc = t_per_w // C
    mesh = plsc.VectorSubcoreMesh(core_axis_name="c", subcore_axis_name="s")

    @functools.partial(
        pl.kernel,
        mesh=mesh,
        out_type=jax.ShapeDtypeStruct((N_ROWS, D), jnp.float32),
        scratch_types=[
            pltpu.VMEM((B, t_per_w), jnp.int32),
            [[pltpu.VMEM((C, D), jnp.float32)] * 2] * 4,
            [pltpu.VMEM((C, D), jnp.float32)] * 2,
            [[pltpu.SemaphoreType.DMA] * 2] * 4,
            [[pltpu.SemaphoreType.DMA] * 2] * 4,
            [pltpu.SemaphoreType.DMA] * 2,
        ],
    )
    def k(x_hbm, we_hbm, pe_hbm, out_hbm, idx_all, rows, pes, sg, so, sp):
        wid = lax.axis_index("s") * nc + lax.axis_index("c")
        t0 = wid * t_per_w

        def gather(bh, i, tc, g):
            b = 2 * bh + i
            return pltpu.make_async_copy(
                we_hbm.at[idx_all.at[b, pl.ds(tc * C, C)]],
                rows[g][i], sg[g][i])

        def out_copy(bh, i, tc, g):
            b = 2 * bh + i
            return pltpu.make_async_copy(
                rows[g][i], out_hbm.at[pl.ds(b * T + t0 + tc * C, C)],
                so[g][i])

        def pe_copy(tc, par):
            return pltpu.make_async_copy(
                pe_hbm.at[pl.ds(t0 + tc * C, C)], pes[par], sp[par])

        for b in range(B):
            pltpu.sync_copy(x_hbm.at[pl.ds(b * T + t0, t_per_w)],
                            idx_all.at[b])
        pe_copy(0, 0).start()
        for i in range(2):
            gather(0, i, 0, 0).start()
        for i in range(2):
            gather(1, i, 0, 1).start()

        def tco_body(tco, _):
            for p in range(4):
                tc = tco * 2 + p // 2
                bh = p % 2
                par = p // 2
                g2 = (p + 2) % 4
                for i in range(2):
                    gather(bh, i, tc, p).wait()
                if bh == 0:
                    pe_copy(tc, par).wait()

                    @pl.when(tc + 1 < n_tc)
                    def _():
                        pe_copy(tc + 1, 1 - par).start()

                @pl.when(tc > 0)
                def _():
                    for i in range(2):
                        out_copy(bh, i, tc - 1, g2).wait()

                @pl.when(tc + 1 < n_tc)
                def _():
                    for i in range(2):
                        gather(bh, i, tc + 1, g2).start()

                def row_body(r, _):
                    for jj in range(D // LANES):
                        sl = pl.ds(jj * LANES, LANES)
                        pe_vec = pes[par][r, sl]
                        for i in range(2):
                            rows[p][i][r, sl] = (
                                rows[p][i][r, sl] * SCALE + pe_vec)
                    return 0

                lax.fori_loop(0, C, row_body, 0)
                for i in range(2):
                    out_copy(bh, i, tc, p).start()
            return 0

        lax.fori_loop(0, n_tc // 2, tco_body, 0)
        for bh in range(2):
            for i in range(2):
                out_copy(bh, i, n_tc - 1, 2 + bh).wait()

    return k


_sc_kernel = _make_sc_kernel()


@jax.jit
def kernel(x, We):
    pe = jnp.asarray(_PE)
    flat_idx = x.reshape(-1).astype(jnp.int32)
    out = _sc_kernel(flat_idx, We, pe)
    return out.reshape(B, T, D)
